# baseline (speedup 1.0000x reference)
import functools

import jax
import jax.numpy as jnp
from jax import lax
from jax.experimental import pallas as pl
from jax.experimental.pallas import tpu as pltpu
from jax.experimental.pallas import tpu_sc as plsc

NBUF = 4
PADW = 128
CB = 4


def _emb_lookup(idx, tpad, B0, S, D):
    B = idx.shape[0]
    NW = 32
    per_w = B // NW
    bpw = B0 // NW
    n_chunks = bpw // CB
    assert bpw % CB == 0 and n_chunks % NBUF == 0 and n_chunks // NBUF >= 2
    assert (CB * S) % 8 == 0

    mesh = plsc.VectorSubcoreMesh(core_axis_name="c", subcore_axis_name="s")

    SP = (S + 7) // 8 * 8

    @functools.partial(
        pl.kernel,
        out_type=jax.ShapeDtypeStruct((B0, SP, PADW), jnp.int32),
        mesh=mesh,
        compiler_params=pltpu.CompilerParams(use_tc_tiling_on_sc=False),
        scratch_types=[
            pltpu.VMEM((per_w,), jnp.int32),
            [pltpu.VMEM((CB * S, D), jnp.int32) for _ in range(NBUF)],
            [pltpu.SemaphoreType.DMA for _ in range(NBUF)],
        ],
    )
    def emb(idx_hbm, tpad_hbm, out_hbm, idx_v, bufs, sems):
        wid = lax.axis_index("s") * 2 + lax.axis_index("c")
        base = wid * bpw
        pltpu.sync_copy(idx_hbm.at[pl.ds(base * S, per_w)], idx_v)

        def gather(j, b):
            pltpu.async_copy(
                tpad_hbm.at[idx_v.at[pl.ds(j * (CB * S), CB * S)]], bufs[b], sems[b]
            )

        def wait_gather(b):
            pltpu.make_async_copy(
                tpad_hbm.at[pl.ds(0, CB * S)], bufs[b], sems[b]
            ).wait()

        def store(j, b):
            for t in range(CB):
                pltpu.sync_copy(
                    bufs[b].at[pl.ds(t * S, S)],
                    out_hbm.at[base + j * CB + t, pl.ds(0, S), pl.ds(0, D)],
                )

        for b in range(NBUF):
            gather(b, b)

        def outer(k, carry):
            j0 = k * NBUF
            for b in range(NBUF):
                j = j0 + b
                wait_gather(b)
                store(j, b)
                gather(j + NBUF, b)
            return carry

        lax.fori_loop(0, n_chunks // NBUF - 1, outer, 0, unroll=False)

        for b in range(NBUF):
            j = n_chunks - NBUF + b
            wait_gather(b)
            store(j, b)

    return emb(idx, tpad)


def kernel(x, table):
    B0, S = x.shape
    V, D = table.shape
    idx = x.reshape(-1).astype(jnp.int32)
    t64 = jax.lax.bitcast_convert_type(table, jnp.int64)
    t32 = jax.lax.bitcast_convert_type(t64, jnp.int32).reshape(V, D)
    opad = _emb_lookup(idx, t32, B0, S, D)
    out_i = opad[:, :S, :D]
    return jax.lax.bitcast_convert_type(out_i, jnp.float32)


# speedup vs baseline: 1.4659x; 1.4659x over previous
import functools

import jax
import jax.numpy as jnp
from jax import lax
from jax.experimental import pallas as pl
from jax.experimental.pallas import tpu as pltpu
from jax.experimental.pallas import tpu_sc as plsc

NBUF = 4
PADW = 128
CB = 4


def _emb_lookup(idx, tpad, B0, S, D):
    B = idx.shape[0]
    NW = 32
    per_w = B // NW
    bpw = B0 // NW
    n_chunks = bpw // CB
    assert bpw % CB == 0 and n_chunks % NBUF == 0 and n_chunks // NBUF >= 2
    assert (CB * S) % 8 == 0

    mesh = plsc.VectorSubcoreMesh(core_axis_name="c", subcore_axis_name="s")

    SP = (S + 7) // 8 * 8

    @functools.partial(
        pl.kernel,
        out_type=jax.ShapeDtypeStruct((B0, SP, PADW), jnp.float32),
        mesh=mesh,
        compiler_params=pltpu.CompilerParams(use_tc_tiling_on_sc=False),
        scratch_types=[
            pltpu.VMEM((per_w,), jnp.int32),
            [pltpu.VMEM((CB * S, PADW), jnp.float32) for _ in range(NBUF)],
            [pltpu.SemaphoreType.DMA for _ in range(NBUF)],
        ],
    )
    def emb(idx_hbm, tpad_hbm, out_hbm, idx_v, bufs, sems):
        wid = lax.axis_index("s") * 2 + lax.axis_index("c")
        base = wid * bpw
        pltpu.sync_copy(idx_hbm.at[pl.ds(base * S, per_w)], idx_v)

        def gather(j, b):
            pltpu.async_copy(
                tpad_hbm.at[idx_v.at[pl.ds(j * (CB * S), CB * S)]], bufs[b], sems[b]
            )

        def wait_gather(b):
            pltpu.make_async_copy(
                tpad_hbm.at[pl.ds(0, CB * S)], bufs[b], sems[b]
            ).wait()

        def store(j, b):
            for t in range(CB):
                pltpu.sync_copy(
                    bufs[b].at[pl.ds(t * S, S), pl.ds(0, D)],
                    out_hbm.at[base + j * CB + t, pl.ds(0, S), pl.ds(0, D)],
                )

        for b in range(NBUF):
            gather(b, b)

        def outer(k, carry):
            j0 = k * NBUF
            for b in range(NBUF):
                j = j0 + b
                wait_gather(b)
                store(j, b)
                gather(j + NBUF, b)
            return carry

        lax.fori_loop(0, n_chunks // NBUF - 1, outer, 0, unroll=False)

        for b in range(NBUF):
            j = n_chunks - NBUF + b
            wait_gather(b)
            store(j, b)

    return emb(idx, tpad)


def kernel(x, table):
    B0, S = x.shape
    V, D = table.shape
    idx = x.reshape(-1).astype(jnp.int32)
    tpad = jnp.pad(table, ((0, 0), (0, PADW - D)))
    opad = _emb_lookup(idx, tpad, B0, S, D)
    return opad[:, :S, :D]
